# Initial kernel scaffold; baseline (speedup 1.0000x reference)
import jax
import jax.numpy as jnp
from jax.experimental import pallas as pl


def kernel(x_coarse, p_rows, p_cols, p_vals):
    raise NotImplementedError("write your pallas kernel here")


# speedup vs baseline: 2.0723x; 2.0723x over previous
import functools

import jax
import jax.numpy as jnp
from jax import lax
from jax.experimental import pallas as pl
from jax.experimental.pallas import tpu as pltpu
from jax.experimental.pallas import tpu_sc as plsc

N_F = 65536
N_C = 16384
NNZ = 262144
F = 256

NW = 32
ROWS_W = N_F // NW
BLK = 256
NBLK = ROWS_W // BLK
NBLK_G = N_F // BLK
K = 64
PAD = 8 * K


def kernel(x_coarse, p_rows, p_cols, p_vals):
    p_rows = p_rows.astype(jnp.int32)
    p_cols = p_cols.astype(jnp.int32)

    edges = jnp.searchsorted(
        p_rows, jnp.arange(0, N_F + 1, BLK, dtype=jnp.int32), side="left"
    ).astype(jnp.int32)
    bnd = (
        jnp.zeros((NBLK_G, 16), jnp.int32)
        .at[:, 0].set(edges[:-1])
        .at[:, 1].set(edges[1:])
    )
    rows_p = jnp.concatenate([p_rows, jnp.zeros((PAD,), jnp.int32)])
    cols_p = jnp.concatenate([p_cols, jnp.zeros((PAD,), jnp.int32)])
    vals_p = jnp.concatenate([p_vals, jnp.zeros((PAD,), jnp.float32)])

    mesh = plsc.VectorSubcoreMesh(core_axis_name="c", subcore_axis_name="s")

    @functools.partial(
        pl.kernel,
        out_type=jax.ShapeDtypeStruct((N_F, F), jnp.float32),
        mesh=mesh,
        scratch_types=[
            pltpu.VMEM((16,), jnp.int32),
            pltpu.VMEM((2, K), jnp.int32),
            pltpu.VMEM((2, K), jnp.int32),
            pltpu.VMEM((2, K), jnp.float32),
            pltpu.VMEM((2, K, F), jnp.float32),
            pltpu.VMEM((BLK, F), jnp.float32),
            pltpu.SemaphoreType.DMA,
            pltpu.SemaphoreType.DMA,
            pltpu.SemaphoreType.DMA,
            pltpu.SemaphoreType.DMA,
            pltpu.SemaphoreType.DMA,
            pltpu.SemaphoreType.DMA,
        ],
    )
    def sc_kernel(x_hbm, rows_hbm, cols_hbm, vals_hbm, bnd_hbm, out_hbm,
                  bnd_v, cidx_v, ridx_v, vals_v, gath_v, acc_v,
                  semc0, semc1, semrv0, semrv1, semg0, semg1):
        wid = lax.axis_index("s") * 2 + lax.axis_index("c")
        semc = (semc0, semc1)
        semrv = (semrv0, semrv1)
        semg = (semg0, semg1)

        def block_body(b, _):
            g = wid * NBLK + b
            row_base = g * BLK
            pltpu.sync_copy(bnd_hbm.at[g], bnd_v)
            bl = bnd_v[...]
            start = bl[0]
            end = bl[1]

            zero16 = jnp.zeros((16,), jnp.float32)

            def zero_body(r, _2):
                for fc in range(F // 16):
                    acc_v[r, pl.ds(fc * 16, 16)] = zero16
                return 0

            lax.fori_loop(0, BLK, zero_body, 0, unroll=False)

            s0 = jnp.bitwise_and(start, jnp.int32(-8))
            nch = (end - s0 + (K - 1)) // K
            npair = (nch + 1) // 2

            def chunk_at(i):
                return pl.multiple_of(s0 + i * K, 8)

            def fire_cols(i, p):
                s = chunk_at(i)
                return pltpu.async_copy(
                    cols_hbm.at[pl.ds(s, K)], cidx_v.at[p], semc[p])

            def fire_rv(i, p):
                s = chunk_at(i)
                pltpu.async_copy(rows_hbm.at[pl.ds(s, K)], ridx_v.at[p],
                                 semrv[p])
                pltpu.async_copy(vals_hbm.at[pl.ds(s, K)], vals_v.at[p],
                                 semrv[p])

            def wait_rv(p):
                pltpu.make_async_copy(rows_hbm.at[pl.ds(0, K)],
                                      ridx_v.at[p], semrv[p]).wait()
                pltpu.make_async_copy(vals_hbm.at[pl.ds(0, K)],
                                      vals_v.at[p], semrv[p]).wait()

            def wait_cols(p):
                pltpu.make_async_copy(cols_hbm.at[pl.ds(0, K)],
                                      cidx_v.at[p], semc[p]).wait()

            def fire_gather(p):
                return pltpu.async_copy(
                    x_hbm.at[cidx_v.at[p]], gath_v.at[p], semg[p])

            def wait_gather(p):
                pltpu.make_async_copy(x_hbm.at[cidx_v.at[p]],
                                      gath_v.at[p], semg[p]).wait()

            def compute(i, p):
                s = chunk_at(i)
                for q in range(K // 16):
                    sl = pl.ds(q * 16, 16)
                    gk = lax.iota(jnp.int32, 16) + (s + q * 16)
                    vv = jnp.where((gk >= start) & (gk < end),
                                   vals_v[p, sl], 0.0)
                    lr = jnp.clip(ridx_v[p, sl] - row_base, 0, BLK - 1)
                    for lane in range(16):
                        kk = q * 16 + lane
                        row_ref = acc_v.at[lr[lane]]
                        vk = vv[lane]
                        for fc in range(F // 16):
                            fsl = pl.ds(fc * 16, 16)
                            plsc.addupdate(row_ref.at[fsl],
                                           gath_v[p, kk, fsl] * vk)

            fire_cols(0, 0)
            fire_rv(0, 0)
            fire_cols(1, 1)
            fire_rv(1, 1)
            wait_cols(0)
            fire_gather(0)

            def pair_body(j, _2):
                i0 = 2 * j
                for sub in range(2):
                    i = i0 + sub
                    p = sub
                    pp = 1 - sub
                    wait_cols(pp)
                    fire_gather(pp)
                    wait_gather(p)
                    fire_cols(i + 2, p)
                    wait_rv(p)
                    compute(i, p)
                    fire_rv(i + 2, p)
                return 0

            lax.fori_loop(0, npair, pair_body, 0, unroll=False)

            wait_cols(1)
            wait_rv(0)
            wait_rv(1)
            wait_gather(0)

            pltpu.sync_copy(acc_v, out_hbm.at[pl.ds(row_base, BLK)])
            return 0

        lax.fori_loop(0, NBLK, block_body, 0, unroll=False)

    return sc_kernel(x_coarse, rows_p, cols_p, vals_p, bnd)
